# baseline (speedup 1.0000x reference)
import jax
import jax.numpy as jnp
from jax import lax
from jax.experimental import pallas as pl
from jax.experimental.pallas import tpu as pltpu
from jax.experimental.pallas import tpu_sc as plsc

ROWS, COLS = 16384, 200
VOCAB, DIM = 100000, 64
PAD = 128
LANES = 16

NC, NS = 2, 16
NW = NC * NS

NB = ROWS // PAD
BPW = NB // NW
NBUF = 2
KB = 2
NCHUNK = COLS * (BPW // KB)
NPAIR = NCHUNK // NBUF

assert BPW == NBUF * KB and NPAIR == COLS


def _emb_body(species_hbm, table_hbm, out_hbm,
              idx_v, rows_v, trans_v, sem_g0, sem_g1, sem_o0, sem_o1):
    wid = lax.axis_index("s") * NC + lax.axis_index("c")
    sem_g = (sem_g0, sem_g1)
    sem_o = (sem_o0, sem_o1)
    iota = lax.iota(jnp.int32, LANES)
    rowvecs = [dd * LANES + iota for dd in range(DIM // LANES)]

    def load_and_fire(r, h, b):
        pltpu.sync_copy(species_hbm.at[r, wid * KB + h], idx_v.at[b])
        for k in range(KB):
            pltpu.async_copy(
                table_hbm.at[idx_v.at[b, pl.ds(k * PAD, PAD)]],
                rows_v.at[b, k],
                sem_g[b],
            )

    def drain_gathers(b):
        for k in range(KB):
            pltpu.make_async_copy(
                table_hbm.at[idx_v.at[b, pl.ds(k * PAD, PAD)]],
                rows_v.at[b, k],
                sem_g[b],
            ).wait()

    def fire_stores(r, h, b):
        for k in range(KB):
            pltpu.async_copy(
                trans_v.at[b, k],
                out_hbm.at[r, :, pl.ds((wid * BPW + h * KB + k) * PAD, PAD)],
                sem_o[b],
            )

    def wait_stores(r, h, b):
        for k in range(KB):
            pltpu.make_async_copy(
                trans_v.at[b, k],
                out_hbm.at[r, :, pl.ds((wid * BPW + h * KB + k) * PAD, PAD)],
                sem_o[b],
            ).wait()

    def transpose(b):
        def j_body(j, carry):
            jsplat = jnp.full((LANES,), j, jnp.int32)
            for k in range(KB):
                for dd in range(DIM // LANES):
                    v = rows_v[b, k, j, pl.ds(dd * LANES, LANES)]
                    plsc.store_scatter(trans_v.at[b, k], [rowvecs[dd], jsplat], v)
            return carry
        lax.fori_loop(0, PAD, j_body, 0)


    load_and_fire(0, 0, 0)
    load_and_fire(0, 1, 1)
    for b in range(NBUF):
        drain_gathers(b)
        transpose(b)
        fire_stores(0, b, b)
        load_and_fire(1, b, b)

    def pair_body(r, carry):
        for b in range(NBUF):
            drain_gathers(b)
            wait_stores(r - 1, b, b)
            transpose(b)
            fire_stores(r, b, b)
            load_and_fire(r + 1, b, b)
        return carry

    lax.fori_loop(1, COLS - 1, pair_body, 0)

    for b in range(NBUF):
        drain_gathers(b)
        wait_stores(COLS - 2, b, b)
        transpose(b)
        fire_stores(COLS - 1, b, b)
        wait_stores(COLS - 1, b, b)


@jax.jit
def _embed(species_half, tablepad):
    mesh = plsc.VectorSubcoreMesh(
        core_axis_name="c", subcore_axis_name="s",
        num_cores=NC, num_subcores=NS)
    run = pl.kernel(
        _emb_body,
        out_type=jax.ShapeDtypeStruct((COLS, DIM, ROWS), jnp.float32),
        mesh=mesh,
        scratch_types=[
            pltpu.VMEM((NBUF, KB * PAD), jnp.int32),
            pltpu.VMEM((NBUF, KB, PAD, PAD), jnp.float32),
            pltpu.VMEM((NBUF, KB, DIM, PAD), jnp.float32),
            pltpu.SemaphoreType.DMA,
            pltpu.SemaphoreType.DMA,
            pltpu.SemaphoreType.DMA,
            pltpu.SemaphoreType.DMA,
        ],
        compiler_params=pltpu.CompilerParams(
            use_tc_tiling_on_sc=True, needs_layout_passes=False),
    )
    return run(species_half, tablepad)


def kernel(species, embedding):
    species_half = species.T.reshape(COLS, NW * NBUF, KB * PAD).astype(jnp.int32)
    tablepad = jnp.pad(embedding, ((0, 0), (0, PAD - DIM)))
    out_t = _embed(species_half, tablepad)
    return jnp.transpose(out_t, (2, 0, 1))
